# baseline (speedup 1.0000x reference)
import functools

import jax
import jax.numpy as jnp
from jax import lax
from jax.experimental import pallas as pl
from jax.experimental.pallas import tpu as pltpu
from jax.experimental.pallas import tpu_sc as plsc

VOCAB = 1000
N_TOK = 1024 * 50
NC, NS, L = 2, 16, 16
NW = NC * NS
NSTRIPE = VOCAB // 8
SPW = 4
BLK = 2048
NBLK = N_TOK // BLK
TILES = BLK // 128
TPW = N_TOK // NW
LROWS = (TPW + 127) // 128


def _lse_body(table_ref, lse_ref):
    x = table_ref[...]
    m = jnp.max(x, axis=1, keepdims=True)
    s = jnp.sum(jnp.exp(x - m), axis=1, keepdims=True)
    lse_ref[...] = jnp.log(s) + m


_lse_call = pl.pallas_call(
    _lse_body,
    out_shape=jax.ShapeDtypeStruct((VOCAB, 1), jnp.float32),
)


def _sc_body(tflat_hbm, idx_hbm, tgt_hbm, lse_hbm, out_hbm, psum_hbm,
             tb0, tb1, tb2, tb3, sg0, sg1, sg2, sg3,
             idxb_v, lse_v, idxo_v, tgto_v, offs_v, vals_v, acc_v,
             dsem0, dsem1, dsem2, dsem3, strsem):
    wid = lax.axis_index("s") * NC + lax.axis_index("c")
    tblk = (tb0, tb1, tb2, tb3)
    stg = (sg0, sg1, sg2, sg3)
    dsem = (dsem0, dsem1, dsem2, dsem3)

    pltpu.sync_copy(lse_hbm, lse_v)
    pltpu.sync_copy(idx_hbm.at[pl.ds(wid * TPW, TPW)], idxo_v)
    pltpu.sync_copy(tgt_hbm.at[pl.ds(wid * TPW, TPW)], tgto_v)
    acc_v[...] = jnp.zeros((L,), jnp.float32)
    for c in range(TPW // L):
        sl = pl.ds(c * L, L)
        i16 = idxo_v[sl]
        offs_v[c // 8, pl.ds((c % 8) * L, L)] = tgto_v[sl] * VOCAB + i16
        acc_v[...] = acc_v[...] + plsc.load_gather(lse_v, [i16])
    for c in range(LROWS * 8 - TPW // L):
        offs_v[LROWS - 1, pl.ds((TPW // L % 8 + c) * L, L)] = (
            jnp.zeros((L,), jnp.int32))
    for r in range(LROWS):
        pltpu.async_copy(tflat_hbm.at[offs_v.at[r]], vals_v.at[r], strsem)
    for r in range(LROWS):
        pltpu.make_async_copy(
            tflat_hbm.at[offs_v.at[0]], vals_v.at[0], strsem).wait()
    for c in range(TPW // L):
        acc_v[...] = acc_v[...] - vals_v[c // 8, pl.ds((c % 8) * L, L)]
    pltpu.sync_copy(acc_v, psum_hbm.at[wid])

    for j in range(SPW):
        sj = wid + NW * j

        @pl.when(sj < NSTRIPE)
        def _(j=j, sj=sj):
            pltpu.sync_copy(tflat_hbm.at[pl.ds(sj * 8 * VOCAB, 8 * VOCAB)],
                            tblk[j])

    def blk_body(b, carry):
        pltpu.sync_copy(idx_hbm.at[pl.ds(b * BLK, BLK)], idxb_v)
        for j in range(SPW):
            sj = wid + NW * j

            @pl.when(sj < NSTRIPE)
            def _(j=j, sj=sj):
                @pl.when(b > 0)
                def _():
                    pltpu.make_async_copy(
                        stg[j], out_hbm.at[sj, pl.ds(0, TILES)],
                        dsem[j]).wait()

                @plsc.parallel_loop(0, TILES, 1, unroll=4)
                def tile_body(t, j=j):
                    for ch in range(8):
                        i16 = idxb_v[pl.ds(t * 128 + ch * L, L)]
                        for cc in range(8):
                            v16 = plsc.load_gather(tblk[j], [i16 + cc * VOCAB])
                            stg[j][t, cc, pl.ds(ch * L, L)] = v16
                pltpu.async_copy(
                    stg[j], out_hbm.at[sj, pl.ds(b * TILES, TILES)], dsem[j])
        return carry

    lax.fori_loop(0, NBLK, blk_body, 0)
    for j in range(SPW):
        sj = wid + NW * j

        @pl.when(sj < NSTRIPE)
        def _(j=j, sj=sj):
            pltpu.make_async_copy(
                stg[j], out_hbm.at[sj, pl.ds(0, TILES)], dsem[j]).wait()


_sc_call = functools.partial(
    pl.kernel,
    mesh=plsc.VectorSubcoreMesh(core_axis_name="c", subcore_axis_name="s"),
    compiler_params=pltpu.CompilerParams(
        use_tc_tiling_on_sc=False, needs_layout_passes=False),
    out_type=[
        jax.ShapeDtypeStruct((NSTRIPE, N_TOK // 128, 8, 128), jnp.float32),
        jax.ShapeDtypeStruct((NW, L), jnp.float32),
    ],
    scratch_types=[
        pltpu.VMEM((8 * VOCAB,), jnp.float32),
        pltpu.VMEM((8 * VOCAB,), jnp.float32),
        pltpu.VMEM((8 * VOCAB,), jnp.float32),
        pltpu.VMEM((8 * VOCAB,), jnp.float32),
        pltpu.VMEM((TILES, 8, 128), jnp.float32),
        pltpu.VMEM((TILES, 8, 128), jnp.float32),
        pltpu.VMEM((TILES, 8, 128), jnp.float32),
        pltpu.VMEM((TILES, 8, 128), jnp.float32),
        pltpu.VMEM((BLK,), jnp.int32),
        pltpu.VMEM((VOCAB,), jnp.float32),
        pltpu.VMEM((TPW,), jnp.int32),
        pltpu.VMEM((TPW,), jnp.int32),
        pltpu.VMEM((LROWS, 128), jnp.int32),
        pltpu.VMEM((LROWS, 128), jnp.float32),
        pltpu.VMEM((L,), jnp.float32),
        pltpu.SemaphoreType.DMA,
        pltpu.SemaphoreType.DMA,
        pltpu.SemaphoreType.DMA,
        pltpu.SemaphoreType.DMA,
        pltpu.SemaphoreType.DMA,
    ],
)(_sc_body)


def _loss_body(psum_ref, out_ref):
    out_ref[...] = jnp.sum(psum_ref[...], keepdims=True) / N_TOK


_loss_call = pl.pallas_call(
    _loss_body,
    out_shape=jax.ShapeDtypeStruct((1, 1), jnp.float32),
)


def kernel(idx, target, embedding_table):
    idxf = idx.reshape(-1).astype(jnp.int32)
    tgtf = target.reshape(-1).astype(jnp.int32)
    table = embedding_table.astype(jnp.float32)
    lse = _lse_call(table).reshape(VOCAB)
    tflat = table.T.reshape(-1)
    out4d, psum = _sc_call(tflat, idxf, tgtf, lse)
    logits = out4d.transpose(1, 3, 0, 2).reshape(N_TOK, VOCAB)
    loss = _loss_call(psum).reshape(())
    return logits, loss
